# Initial kernel scaffold; baseline (speedup 1.0000x reference)
import jax
import jax.numpy as jnp
from jax.experimental import pallas as pl


def kernel(x, edge_index, symbol_ids, lengths, W_msg, W_self, b_gnn, W1, b1, W2, b2):
    raise NotImplementedError("write your pallas kernel here")


# speedup vs baseline: 6.4549x; 6.4549x over previous
import functools

import jax
import jax.numpy as jnp
from jax import lax
from jax.experimental import pallas as pl
from jax.experimental.pallas import tpu as pltpu
from jax.experimental.pallas import tpu_sc as plsc

_NC = 2
_NS = 16
_L = 16

_SC_PARAMS = pltpu.CompilerParams(needs_layout_passes=False)


def _edge_agg(x, src, dst):
    N, D = x.shape
    E = src.shape[0]
    CH = 128
    n_chunks = E // CH
    assert n_chunks * CH == E
    nw = _NC * _NS
    base_full = n_chunks // nw
    rem = n_chunks - base_full * nw
    RPS = 624
    TAIL = N - RPS * (_NS - 1)
    ZB = 208
    assert RPS % ZB == 0 and (TAIL - RPS) % 8 == 0

    mesh = plsc.VectorSubcoreMesh(core_axis_name="c", subcore_axis_name="s")

    @functools.partial(
        pl.kernel,
        out_type=jax.ShapeDtypeStruct((_NC, N, D), jnp.float32),
        mesh=mesh,
        scratch_types=[
            pltpu.VMEM((1, CH), jnp.int32),
            pltpu.VMEM((1, CH), jnp.int32),
            pltpu.VMEM((CH, D), jnp.float32),
            pltpu.VMEM((ZB, D), jnp.float32),
            pltpu.VMEM_SHARED((N, D), jnp.float32),
        ],
        compiler_params=_SC_PARAMS,
    )
    def k(x_hbm, src_hbm, dst_hbm, out_hbm, sidx, didx, rows, zbuf, agg):
        cid = lax.axis_index("c")
        sid = lax.axis_index("s")
        gwid = cid * _NS + sid

        @pl.loop(0, ZB)
        def _(i):
            for j in range(D // _L):
                zbuf[i, pl.ds(j * _L, _L)] = jnp.zeros((_L,), jnp.float32)

        for kz in range(RPS // ZB):
            pltpu.sync_copy(zbuf, agg.at[pl.ds(sid * RPS + kz * ZB, ZB)])

        @pl.when(sid == _NS - 1)
        def _():
            pltpu.sync_copy(zbuf.at[pl.ds(0, TAIL - RPS)],
                            agg.at[pl.ds(_NS * RPS, TAIL - RPS)])
        plsc.subcore_barrier()

        base = base_full * gwid + jnp.minimum(gwid, rem)

        def do_chunk(g):
            off = g * CH
            pltpu.sync_copy(src_hbm.at[pl.ds(off, CH)], sidx.at[0])
            pltpu.sync_copy(dst_hbm.at[pl.ds(off, CH)], didx.at[0])
            pltpu.sync_copy(x_hbm.at[sidx.at[0]], rows)
            pltpu.sync_copy(rows, agg.at[didx.at[0]], add=True)

        @pl.loop(0, base_full)
        def _(c):
            do_chunk(base + c)

        @pl.when(gwid < rem)
        def _():
            do_chunk(base + base_full)

        plsc.subcore_barrier()
        for kz in range(RPS // ZB):
            sl = pl.ds(sid * RPS + kz * ZB, ZB)
            pltpu.sync_copy(agg.at[sl], out_hbm.at[cid].at[sl])

        @pl.when(sid == _NS - 1)
        def _():
            sl = pl.ds(_NS * RPS, TAIL - RPS)
            pltpu.sync_copy(agg.at[sl], out_hbm.at[cid].at[sl])

    return k(x, src, dst)


def _node_update(parts, x, W_msg, W_self, b_gnn2d):
    N, D = x.shape
    BLK = 1000

    def body(p_ref, x_ref, wm_ref, ws_ref, b_ref, o_ref):
        agg = p_ref[0] + p_ref[1]
        acc = jnp.dot(agg, wm_ref[...], preferred_element_type=jnp.float32)
        acc = acc + jnp.dot(x_ref[...], ws_ref[...],
                            preferred_element_type=jnp.float32)
        o_ref[...] = jnp.maximum(acc + b_ref[...], 0.0)

    return pl.pallas_call(
        body,
        grid=(N // BLK,),
        in_specs=[
            pl.BlockSpec((_NC, BLK, D), lambda i: (0, i, 0)),
            pl.BlockSpec((BLK, D), lambda i: (i, 0)),
            pl.BlockSpec((D, D), lambda i: (0, 0)),
            pl.BlockSpec((D, D), lambda i: (0, 0)),
            pl.BlockSpec((1, D), lambda i: (0, 0)),
        ],
        out_specs=pl.BlockSpec((BLK, D), lambda i: (i, 0)),
        out_shape=jax.ShapeDtypeStruct((N, D), jnp.float32),
    )(parts, x, W_msg, W_self, b_gnn2d)


def _token_sum(h, ids_flat, lengths, T):
    N, D = h.shape
    B = lengths.shape[0]
    nw = _NC * _NS
    TH = T // 2
    W = 128
    nblk = TH // W
    assert B * 2 == nw and nblk * W == TH

    mesh = plsc.VectorSubcoreMesh(core_axis_name="c", subcore_axis_name="s")

    @functools.partial(
        pl.kernel,
        out_type=jax.ShapeDtypeStruct((nw * D,), jnp.float32),
        mesh=mesh,
        scratch_types=[
            pltpu.VMEM((1, W), jnp.int32),
            pltpu.VMEM((W, D), jnp.float32),
            pltpu.VMEM((1, D), jnp.float32),
            pltpu.VMEM((1, B), jnp.int32),
        ],
        compiler_params=_SC_PARAMS,
    )
    def k(h_hbm, ids_hbm, len_hbm, out_hbm, idx, rows, accbuf, len_v):
        cid = lax.axis_index("c")
        sid = lax.axis_index("s")
        gwid = cid * _NS + sid
        b = gwid // 2
        t0 = (gwid % 2) * TH
        pltpu.sync_copy(len_hbm, len_v.at[0])
        lane = lax.iota(jnp.int32, _L)
        lb = jnp.sum(jnp.where(lane == b, len_v[0, pl.ds(0, _L)], 0))
        cnt = jnp.clip(lb - 1 - t0, 0, TH)

        for j in range(D // _L):
            accbuf[0, pl.ds(j * _L, _L)] = jnp.zeros((_L,), jnp.float32)

        for blk in range(nblk):
            @pl.when(cnt > blk * W)
            def _():
                pltpu.sync_copy(ids_hbm.at[pl.ds(b * T + t0 + blk * W, W)],
                                idx.at[0])
                pltpu.sync_copy(h_hbm.at[idx.at[0]], rows)

                @pl.loop(0, W)
                def _(t):
                    @pl.when(blk * W + t < cnt)
                    def _():
                        for j in range(D // _L):
                            sl = pl.ds(j * _L, _L)
                            accbuf[0, sl] = accbuf[0, sl] + rows[t, sl]

        pltpu.sync_copy(accbuf.at[0], out_hbm.at[pl.ds(gwid * D, D)])

    return k(h, ids_flat, lengths)


def _head(parts, W1, b1_2d, W2, b2_2d):
    B = parts.shape[0]
    D = parts.shape[2]
    O = W2.shape[1]

    def body(p_ref, w1_ref, b1_ref, w2_ref, b2_ref, o_ref):
        summed = p_ref[:, 0, :] + p_ref[:, 1, :]
        hid = jnp.maximum(
            jnp.dot(summed, w1_ref[...], preferred_element_type=jnp.float32)
            + b1_ref[...], 0.0)
        o_ref[...] = (jnp.dot(hid, w2_ref[...],
                              preferred_element_type=jnp.float32)
                      + b2_ref[...])

    return pl.pallas_call(
        body,
        out_shape=jax.ShapeDtypeStruct((B, O), jnp.float32),
    )(parts, W1, b1_2d, W2, b2_2d)


def kernel(x, edge_index, symbol_ids, lengths, W_msg, W_self, b_gnn,
           W1, b1, W2, b2):
    src = edge_index[0]
    dst = edge_index[1]
    parts = _edge_agg(x, src, dst)
    h = _node_update(parts, x, W_msg, W_self, b_gnn.reshape(1, -1))
    T = symbol_ids.shape[0]
    ids_flat = symbol_ids.T.reshape(-1)
    pooled = _token_sum(h, ids_flat, lengths, T)
    pooled = pooled.reshape(lengths.shape[0], 2, x.shape[1])
    return _head(pooled, W1, b1.reshape(1, -1), W2, b2.reshape(1, -1))
